# baseline (speedup 1.0000x reference)
import functools
import math

import jax
import jax.numpy as jnp
from jax import lax
from jax.experimental import pallas as pl
from jax.experimental.pallas import tpu as pltpu
from jax.experimental.pallas import tpu_sc as plsc

VOCAB = 1000000
D = 64
B = 4096
L = 200
N = B * L

NC = 2
NS = 16
NW = NC * NS
CHUNK = 128
N_CHUNKS = L
NBUF = 5

SCALE = math.sqrt(D)

_mesh = plsc.VectorSubcoreMesh(core_axis_name="c", subcore_axis_name="s")


@functools.partial(
    pl.kernel,
    mesh=_mesh,
    out_type=jax.ShapeDtypeStruct((L, D // 8, B // CHUNK, 8, CHUNK), jnp.float32),
    scratch_types=[
        pltpu.VMEM((L, CHUNK), jnp.int32),
        pltpu.VMEM((NBUF, CHUNK, D), jnp.float32),
        pltpu.VMEM((NBUF, D // 8, 8, CHUNK), jnp.float32),
    ]
    + [pltpu.SemaphoreType.DMA] * (2 * NBUF),
    compiler_params=pltpu.CompilerParams(
        use_tc_tiling_on_sc=False, needs_layout_passes=False
    ),
)
def _emb_lookup(xt_hbm, table_hbm, out_hbm, idx_v, rows_v, tpose_v, *sems):
    gsems = sems[:NBUF]
    osems = sems[NBUF:]
    wid = lax.axis_index("s") * NC + lax.axis_index("c")
    b0 = wid * CHUNK

    pltpu.sync_copy(xt_hbm.at[:, pl.ds(b0, CHUNK)], idx_v)

    def g_copy(c, b):
        return pltpu.make_async_copy(
            table_hbm.at[idx_v.at[c]],
            rows_v.at[b],
            gsems[b],
        )

    def w_copy(c, b):
        return pltpu.make_async_copy(
            tpose_v.at[b],
            out_hbm.at[c, :, wid],
            osems[b],
        )

    iota = lax.broadcasted_iota(jnp.int32, (16,), 0)
    rvecs = [j * 16 + iota for j in range(8)]

    def transpose_scale(b):
        bvec = jnp.full((16,), b, jnp.int32)

        @plsc.parallel_loop(0, D, 1, unroll=2)
        def _t(d):
            dvec = jnp.full((16,), d, jnp.int32)
            dt = d // 8
            ds_ = d % 8
            for j in range(8):
                vals = plsc.load_gather(rows_v, [bvec, rvecs[j], dvec])
                tpose_v[b, dt, ds_, pl.ds(j * 16, 16)] = vals * jnp.float32(SCALE)

    def step(c, par, fire, drain_wb):
        b = par
        nb = (par + NBUF - 1) % NBUF
        g_copy(c, b).wait()
        transpose_scale(b)
        w_copy(c, b).start()
        if fire:
            if drain_wb:
                w_copy(c - 1, nb).wait()
            g_copy(c + NBUF - 1, nb).start()

    for c0 in range(NBUF - 1):
        g_copy(c0, c0).start()

    step(0, 0, fire=True, drain_wb=False)
    for par in range(1, NBUF):
        step(par, par, fire=True, drain_wb=True)

    def outer(go, carry):
        for par in range(NBUF):
            step(go * NBUF + par, par, fire=True, drain_wb=True)
        return carry

    lax.fori_loop(1, N_CHUNKS // NBUF - 1, outer, 0)

    for par in range(NBUF):
        c = (N_CHUNKS - NBUF) + par
        step(c, par, fire=(par == 0), drain_wb=(par == 0))

    for par in range(NBUF):
        w_copy(N_CHUNKS - NBUF + par, par).wait()


def kernel(x, table):
    out5 = _emb_lookup(x.T.astype(jnp.int32), table)
    return out5.transpose(2, 4, 0, 1, 3).reshape(B, L, D)


# speedup vs baseline: 1.1367x; 1.1367x over previous
import functools
import math

import jax
import jax.numpy as jnp
from jax import lax
from jax.experimental import pallas as pl
from jax.experimental.pallas import tpu as pltpu
from jax.experimental.pallas import tpu_sc as plsc

VOCAB = 1000000
D = 64
B = 4096
L = 200
N = B * L
DPAD = 128

NC = 2
NS = 16
NW = NC * NS
CHUNK = 128
N_CHUNKS = L
NBUF = 4
TPITCH = 133

SCALE = math.sqrt(D)


TBLK = 1024


def _tpose_body(t_ref, o_ref):
    x = t_ref[...]
    ii = lax.broadcasted_iota(jnp.int32, (D, DPAD), 0)
    jj = lax.broadcasted_iota(jnp.int32, (D, DPAD), 1)
    eye2 = jnp.where(jj % D == ii, jnp.float32(1.0), jnp.float32(0.0))
    o_ref[...] = lax.dot_general(
        x, eye2, (((0,), (0,)), ((), ())),
        preferred_element_type=jnp.float32,
        precision=lax.Precision.HIGHEST,
    )


def _table_pad_transpose(table_t):
    return pl.pallas_call(
        _tpose_body,
        grid=((VOCAB + TBLK - 1) // TBLK,),
        in_specs=[pl.BlockSpec((D, TBLK), lambda i: (0, i))],
        out_specs=pl.BlockSpec((TBLK, DPAD), lambda i: (i, 0)),
        out_shape=jax.ShapeDtypeStruct((VOCAB, DPAD), jnp.float32),
    )(table_t)


_mesh = plsc.VectorSubcoreMesh(core_axis_name="c", subcore_axis_name="s")


@functools.partial(
    pl.kernel,
    mesh=_mesh,
    out_type=jax.ShapeDtypeStruct((L, D // 8, B // CHUNK, 8, CHUNK), jnp.float32),
    scratch_types=[
        pltpu.VMEM((L, CHUNK), jnp.int32),
        pltpu.VMEM((NBUF, CHUNK, DPAD), jnp.float32),
        pltpu.VMEM((NBUF, D // 8, 8, TPITCH), jnp.float32),
    ]
    + [pltpu.SemaphoreType.DMA] * (2 * NBUF),
    compiler_params=pltpu.CompilerParams(
        use_tc_tiling_on_sc=False, needs_layout_passes=False
    ),
)
def _emb_lookup(xt_hbm, table_hbm, out_hbm, idx_v, rows_v, tpose_v, *sems):
    gsems = sems[:NBUF]
    osems = sems[NBUF:]
    wid = lax.axis_index("s") * NC + lax.axis_index("c")
    b0 = wid * CHUNK

    pltpu.sync_copy(xt_hbm.at[:, pl.ds(b0, CHUNK)], idx_v)

    def g_copy(c, b):
        return pltpu.make_async_copy(
            table_hbm.at[idx_v.at[c]],
            rows_v.at[b],
            gsems[b],
        )

    def w_copy(c, b):
        return pltpu.make_async_copy(
            tpose_v.at[b, :, :, pl.ds(0, CHUNK)],
            out_hbm.at[c, :, wid],
            osems[b],
        )

    iota = lax.broadcasted_iota(jnp.int32, (16,), 0)
    dts = [dg * 2 + iota // 8 for dg in range(4)]
    dsv = iota % 8

    def transpose_scale(b):
        @plsc.parallel_loop(0, CHUNK, 1, unroll=4)
        def _t(r):
            rvec = jnp.full((16,), r, jnp.int32)
            for dg in range(4):
                vals = rows_v[b, r, pl.ds(dg * 16, 16)] * jnp.float32(SCALE)
                plsc.store_scatter(tpose_v.at[b], [dts[dg], dsv, rvec], vals)

    def step(c, par, fire, drain_wb):
        b = par
        nb = (par + NBUF - 1) % NBUF
        g_copy(c, b).wait()
        transpose_scale(b)
        w_copy(c, b).start()
        if fire:
            if drain_wb:
                w_copy(c - 1, nb).wait()
            g_copy(c + NBUF - 1, nb).start()

    for c0 in range(NBUF - 1):
        g_copy(c0, c0).start()

    step(0, 0, fire=True, drain_wb=False)
    for par in range(1, NBUF):
        step(par, par, fire=True, drain_wb=True)

    def outer(go, carry):
        for par in range(NBUF):
            step(go * NBUF + par, par, fire=True, drain_wb=True)
        return carry

    lax.fori_loop(1, N_CHUNKS // NBUF - 1, outer, 0)

    for par in range(NBUF):
        c = (N_CHUNKS - NBUF) + par
        step(c, par, fire=(par == 0), drain_wb=(par == 0))

    for par in range(NBUF):
        w_copy(N_CHUNKS - NBUF + par, par).wait()


def kernel(x, table):
    tpad = _table_pad_transpose(table.T)
    out5 = _emb_lookup(x.T.astype(jnp.int32), tpad)
    return out5.transpose(2, 4, 0, 1, 3).reshape(B, L, D)


# speedup vs baseline: 1.2593x; 1.1079x over previous
import functools
import math

import jax
import jax.numpy as jnp
from jax import lax
from jax.experimental import pallas as pl
from jax.experimental.pallas import tpu as pltpu
from jax.experimental.pallas import tpu_sc as plsc

VOCAB = 1000000
D = 64
B = 4096
L = 200
N = B * L
DPAD = 128

NC = 2
NS = 16
NW = NC * NS
CHUNK = 128
N_CHUNKS = L
NBUF = 4
TPITCH = 133

SCALE = math.sqrt(D)


TBLK = 1024


def _tpose_body(t_ref, o_ref):
    t = t_ref[...].T
    o_ref[...] = jnp.concatenate([t, t], axis=1)


def _table_pad_transpose(table_t):
    return pl.pallas_call(
        _tpose_body,
        grid=((VOCAB + TBLK - 1) // TBLK,),
        in_specs=[pl.BlockSpec((D, TBLK), lambda i: (0, i))],
        out_specs=pl.BlockSpec((TBLK, DPAD), lambda i: (i, 0)),
        out_shape=jax.ShapeDtypeStruct((VOCAB, DPAD), jnp.float32),
    )(table_t)


_mesh = plsc.VectorSubcoreMesh(core_axis_name="c", subcore_axis_name="s")


@functools.partial(
    pl.kernel,
    mesh=_mesh,
    out_type=jax.ShapeDtypeStruct((L, D // 8, B // CHUNK, 8, CHUNK), jnp.float32),
    scratch_types=[
        pltpu.VMEM((L, CHUNK), jnp.int32),
        pltpu.VMEM((NBUF, CHUNK, DPAD), jnp.float32),
        pltpu.VMEM((NBUF, D // 8, 8, TPITCH), jnp.float32),
    ]
    + [pltpu.SemaphoreType.DMA] * (2 * NBUF),
    compiler_params=pltpu.CompilerParams(
        use_tc_tiling_on_sc=False, needs_layout_passes=False
    ),
)
def _emb_lookup(xt_hbm, table_hbm, out_hbm, idx_v, rows_v, tpose_v, *sems):
    gsems = sems[:NBUF]
    osems = sems[NBUF:]
    wid = lax.axis_index("s") * NC + lax.axis_index("c")
    b0 = wid * CHUNK

    pltpu.sync_copy(xt_hbm.at[:, pl.ds(b0, CHUNK)], idx_v)

    def g_copy(c, b):
        return pltpu.make_async_copy(
            table_hbm.at[idx_v.at[c]],
            rows_v.at[b],
            gsems[b],
        )

    def w_copy(c, b):
        return pltpu.make_async_copy(
            tpose_v.at[b, :, :, pl.ds(0, CHUNK)],
            out_hbm.at[c, :, wid],
            osems[b],
        )

    iota = lax.broadcasted_iota(jnp.int32, (16,), 0)
    dts = [dg * 2 + iota // 8 for dg in range(4)]
    dsv = iota % 8

    def transpose_scale(b):
        @plsc.parallel_loop(0, CHUNK, 1, unroll=4)
        def _t(r):
            rvec = jnp.full((16,), r, jnp.int32)
            for dg in range(4):
                vals = rows_v[b, r, pl.ds(dg * 16, 16)] * jnp.float32(SCALE)
                plsc.store_scatter(tpose_v.at[b], [dts[dg], dsv, rvec], vals)

    def step(c, par, fire, drain_wb):
        b = par
        nb = (par + NBUF - 1) % NBUF
        g_copy(c, b).wait()
        transpose_scale(b)
        w_copy(c, b).start()
        if fire:
            if drain_wb:
                w_copy(c - 1, nb).wait()
            g_copy(c + NBUF - 1, nb).start()

    for c0 in range(NBUF - 1):
        g_copy(c0, c0).start()

    step(0, 0, fire=True, drain_wb=False)
    for par in range(1, NBUF):
        step(par, par, fire=True, drain_wb=True)

    def outer(go, carry):
        for par in range(NBUF):
            step(go * NBUF + par, par, fire=True, drain_wb=True)
        return carry

    lax.fori_loop(1, N_CHUNKS // NBUF - 1, outer, 0)

    for par in range(NBUF):
        c = (N_CHUNKS - NBUF) + par
        step(c, par, fire=(par == 0), drain_wb=(par == 0))

    for par in range(NBUF):
        w_copy(N_CHUNKS - NBUF + par, par).wait()


def kernel(x, table):
    tpad = _table_pad_transpose(table.T)
    out5 = _emb_lookup(x.T.astype(jnp.int32), tpad)
    return out5.transpose(2, 4, 0, 1, 3).reshape(B, L, D)


# speedup vs baseline: 2.0153x; 1.6003x over previous
import functools
import math

import jax
import jax.numpy as jnp
from jax import lax
from jax.experimental import pallas as pl
from jax.experimental.pallas import tpu as pltpu
from jax.experimental.pallas import tpu_sc as plsc

VOCAB = 1000000
D = 64
B = 4096
L = 200
N = B * L
DPAD = 128

NC = 2
NS = 16
NW = NC * NS
CHUNK = 128
N_CHUNKS = L
NBUF = 4
TPITCH = 133

SCALE = math.sqrt(D)


TBLK = 4096


def _tpose_body(t_ref, o_ref):
    t = t_ref[...].T
    o_ref[...] = jnp.concatenate([t, t], axis=1)


def _table_pad_transpose(table_t):
    return pl.pallas_call(
        _tpose_body,
        grid=((VOCAB + TBLK - 1) // TBLK,),
        in_specs=[pl.BlockSpec((D, TBLK), lambda i: (0, i))],
        out_specs=pl.BlockSpec((TBLK, DPAD), lambda i: (i, 0)),
        out_shape=jax.ShapeDtypeStruct((VOCAB, DPAD), jnp.float32),
    )(table_t)


_mesh = plsc.VectorSubcoreMesh(core_axis_name="c", subcore_axis_name="s")


@functools.partial(
    pl.kernel,
    mesh=_mesh,
    out_type=jax.ShapeDtypeStruct((L, D // 8, B // CHUNK, 8, CHUNK), jnp.float32),
    scratch_types=[
        pltpu.VMEM((L, CHUNK), jnp.int32),
        pltpu.VMEM((NBUF, CHUNK, DPAD), jnp.float32),
        pltpu.VMEM((NBUF, D // 8, 8, TPITCH), jnp.float32),
    ]
    + [pltpu.SemaphoreType.DMA] * (2 * NBUF),
    compiler_params=pltpu.CompilerParams(
        use_tc_tiling_on_sc=False, needs_layout_passes=False
    ),
)
def _emb_lookup(xt_hbm, table_hbm, out_hbm, idx_v, rows_v, tpose_v, *sems):
    gsems = sems[:NBUF]
    osems = sems[NBUF:]
    wid = lax.axis_index("s") * NC + lax.axis_index("c")
    b0 = wid * CHUNK

    pltpu.sync_copy(xt_hbm.at[:, pl.ds(b0, CHUNK)], idx_v)

    def g_copy(c, b):
        return pltpu.make_async_copy(
            table_hbm.at[idx_v.at[c]],
            rows_v.at[b],
            gsems[b],
        )

    def w_copy(c, b):
        return pltpu.make_async_copy(
            tpose_v.at[b, :, :, pl.ds(0, CHUNK)],
            out_hbm.at[c, :, wid],
            osems[b],
        )

    iota = lax.broadcasted_iota(jnp.int32, (16,), 0)
    dts = [dg * 2 + iota // 8 for dg in range(4)]
    dsv = iota % 8

    def transpose_scale(b):
        @plsc.parallel_loop(0, CHUNK, 1, unroll=4)
        def _t(r):
            rvec = jnp.full((16,), r, jnp.int32)
            for dg in range(4):
                vals = rows_v[b, r, pl.ds(dg * 16, 16)] * jnp.float32(SCALE)
                plsc.store_scatter(tpose_v.at[b], [dts[dg], dsv, rvec], vals)

    def step(c, par, fire, drain_wb):
        b = par
        nb = (par + NBUF - 1) % NBUF
        g_copy(c, b).wait()
        transpose_scale(b)
        w_copy(c, b).start()
        if fire:
            if drain_wb:
                w_copy(c - 1, nb).wait()
            g_copy(c + NBUF - 1, nb).start()

    for c0 in range(NBUF - 1):
        g_copy(c0, c0).start()

    step(0, 0, fire=True, drain_wb=False)
    for par in range(1, NBUF):
        step(par, par, fire=True, drain_wb=True)

    def outer(go, carry):
        for par in range(NBUF):
            step(go * NBUF + par, par, fire=True, drain_wb=True)
        return carry

    lax.fori_loop(1, N_CHUNKS // NBUF - 1, outer, 0)

    for par in range(NBUF):
        c = (N_CHUNKS - NBUF) + par
        step(c, par, fire=(par == 0), drain_wb=(par == 0))

    for par in range(NBUF):
        w_copy(N_CHUNKS - NBUF + par, par).wait()


def kernel(x, table):
    tpad = _table_pad_transpose(table.T)
    out5 = _emb_lookup(x.T.astype(jnp.int32), tpad)
    return out5.transpose(2, 4, 0, 1, 3).reshape(B, L, D)


# speedup vs baseline: 2.4090x; 1.1954x over previous
import functools
import math

import jax
import jax.numpy as jnp
from jax import lax
from jax.experimental import pallas as pl
from jax.experimental.pallas import tpu as pltpu
from jax.experimental.pallas import tpu_sc as plsc

VOCAB = 1000000
D = 64
B = 4096
L = 200
N = B * L
DPAD = 128

NC = 2
NS = 16
NW = NC * NS
CHUNK = 128
N_CHUNKS = L
NBUF = 4
TPITCH = 133

SCALE = math.sqrt(D)


TBLK = 16384


def _tpose_body(t_ref, o_ref):
    t = t_ref[...].T
    o_ref[...] = jnp.concatenate([t, t], axis=1)


def _table_pad_transpose(table_t):
    return pl.pallas_call(
        _tpose_body,
        grid=((VOCAB + TBLK - 1) // TBLK,),
        in_specs=[pl.BlockSpec((D, TBLK), lambda i: (0, i))],
        out_specs=pl.BlockSpec((TBLK, DPAD), lambda i: (i, 0)),
        out_shape=jax.ShapeDtypeStruct((VOCAB, DPAD), jnp.float32),
    )(table_t)


_mesh = plsc.VectorSubcoreMesh(core_axis_name="c", subcore_axis_name="s")


@functools.partial(
    pl.kernel,
    mesh=_mesh,
    out_type=jax.ShapeDtypeStruct((L, D // 8, B // CHUNK, 8, CHUNK), jnp.float32),
    scratch_types=[
        pltpu.VMEM((L, CHUNK), jnp.int32),
        pltpu.VMEM((NBUF, CHUNK, DPAD), jnp.float32),
        pltpu.VMEM((NBUF, D // 8, 8, TPITCH), jnp.float32),
    ]
    + [pltpu.SemaphoreType.DMA] * (2 * NBUF),
    compiler_params=pltpu.CompilerParams(
        use_tc_tiling_on_sc=False, needs_layout_passes=False
    ),
)
def _emb_lookup(xt_hbm, table_hbm, out_hbm, idx_v, rows_v, tpose_v, *sems):
    gsems = sems[:NBUF]
    osems = sems[NBUF:]
    wid = lax.axis_index("s") * NC + lax.axis_index("c")
    b0 = wid * CHUNK

    pltpu.sync_copy(xt_hbm.at[:, pl.ds(b0, CHUNK)], idx_v)

    def g_copy(c, b):
        return pltpu.make_async_copy(
            table_hbm.at[idx_v.at[c]],
            rows_v.at[b],
            gsems[b],
        )

    def w_copy(c, b):
        return pltpu.make_async_copy(
            tpose_v.at[b, :, :, pl.ds(0, CHUNK)],
            out_hbm.at[c, :, wid],
            osems[b],
        )

    iota = lax.broadcasted_iota(jnp.int32, (16,), 0)
    dts = [dg * 2 + iota // 8 for dg in range(4)]
    dsv = iota % 8

    def transpose_scale(b):
        @plsc.parallel_loop(0, CHUNK, 1, unroll=4)
        def _t(r):
            rvec = jnp.full((16,), r, jnp.int32)
            for dg in range(4):
                vals = rows_v[b, r, pl.ds(dg * 16, 16)] * jnp.float32(SCALE)
                plsc.store_scatter(tpose_v.at[b], [dts[dg], dsv, rvec], vals)

    def step(c, par, fire, drain_wb):
        b = par
        nb = (par + NBUF - 1) % NBUF
        g_copy(c, b).wait()
        transpose_scale(b)
        w_copy(c, b).start()
        if fire:
            if drain_wb:
                w_copy(c - 1, nb).wait()
            g_copy(c + NBUF - 1, nb).start()

    for c0 in range(NBUF - 1):
        g_copy(c0, c0).start()

    step(0, 0, fire=True, drain_wb=False)
    for par in range(1, NBUF):
        step(par, par, fire=True, drain_wb=True)

    def outer(go, carry):
        for par in range(NBUF):
            step(go * NBUF + par, par, fire=True, drain_wb=True)
        return carry

    lax.fori_loop(1, N_CHUNKS // NBUF - 1, outer, 0)

    for par in range(NBUF):
        c = (N_CHUNKS - NBUF) + par
        step(c, par, fire=(par == 0), drain_wb=(par == 0))

    for par in range(NBUF):
        w_copy(N_CHUNKS - NBUF + par, par).wait()


def kernel(x, table):
    tpad = _table_pad_transpose(table.T)
    out5 = _emb_lookup(x.T.astype(jnp.int32), tpad)
    return out5.transpose(2, 4, 0, 1, 3).reshape(B, L, D)


# speedup vs baseline: 2.4118x; 1.0011x over previous
import functools
import math

import jax
import jax.numpy as jnp
from jax import lax
from jax.experimental import pallas as pl
from jax.experimental.pallas import tpu as pltpu
from jax.experimental.pallas import tpu_sc as plsc

VOCAB = 1000000
D = 64
B = 4096
L = 200
N = B * L
DPAD = 128

NC = 2
NS = 16
NW = NC * NS
CHUNK = 128
N_CHUNKS = L
NBUF = 4
TPITCH = 133

SCALE = math.sqrt(D)


TBLK = 16384


def _tpose_body(t_ref, o_ref):
    t = t_ref[...].T
    o_ref[...] = jnp.concatenate([t, t], axis=1)


def _table_pad_transpose(table_t):
    return pl.pallas_call(
        _tpose_body,
        grid=((VOCAB + TBLK - 1) // TBLK,),
        in_specs=[pl.BlockSpec((D, TBLK), lambda i: (0, i))],
        out_specs=pl.BlockSpec((TBLK, DPAD), lambda i: (i, 0)),
        out_shape=jax.ShapeDtypeStruct((VOCAB, DPAD), jnp.float32),
    )(table_t)


_mesh = plsc.VectorSubcoreMesh(core_axis_name="c", subcore_axis_name="s")


@functools.partial(
    pl.kernel,
    mesh=_mesh,
    out_type=jax.ShapeDtypeStruct((L, D // 8, B // CHUNK, 8, CHUNK), jnp.float32),
    scratch_types=[
        pltpu.VMEM((L, CHUNK), jnp.int32),
        pltpu.VMEM((NBUF, CHUNK, DPAD), jnp.float32),
        pltpu.VMEM((NBUF, D // 8, 8, TPITCH), jnp.float32),
    ]
    + [pltpu.SemaphoreType.DMA] * (2 * NBUF),
    compiler_params=pltpu.CompilerParams(
        use_tc_tiling_on_sc=False, needs_layout_passes=False
    ),
)
def _emb_lookup(xt_hbm, table_hbm, out_hbm, idx_v, rows_v, tpose_v, *sems):
    gsems = sems[:NBUF]
    osems = sems[NBUF:]
    wid = lax.axis_index("s") * NC + lax.axis_index("c")
    b0 = wid * CHUNK

    pltpu.sync_copy(xt_hbm.at[:, pl.ds(b0, CHUNK)], idx_v)

    def g_copy(c, b):
        return pltpu.make_async_copy(
            table_hbm.at[idx_v.at[c]],
            rows_v.at[b],
            gsems[b],
        )

    def w_copy(c, b):
        return pltpu.make_async_copy(
            tpose_v.at[b, :, :, pl.ds(0, CHUNK)],
            out_hbm.at[c, :, wid],
            osems[b],
        )

    iota = lax.broadcasted_iota(jnp.int32, (16,), 0)
    dts = [dg * 2 + iota // 8 for dg in range(4)]
    dsv = iota % 8

    def transpose_scale(b):
        @plsc.parallel_loop(0, CHUNK, 1, unroll=8)
        def _t(r):
            rvec = jnp.full((16,), r, jnp.int32)
            for dg in range(4):
                vals = rows_v[b, r, pl.ds(dg * 16, 16)] * jnp.float32(SCALE)
                plsc.store_scatter(tpose_v.at[b], [dts[dg], dsv, rvec], vals)

    def step(c, par, fire, drain_wb):
        b = par
        nb = (par + NBUF - 1) % NBUF
        g_copy(c, b).wait()
        transpose_scale(b)
        w_copy(c, b).start()
        if fire:
            if drain_wb:
                w_copy(c - 1, nb).wait()
            g_copy(c + NBUF - 1, nb).start()

    for c0 in range(NBUF - 1):
        g_copy(c0, c0).start()

    step(0, 0, fire=True, drain_wb=False)
    for par in range(1, NBUF):
        step(par, par, fire=True, drain_wb=True)

    def outer(go, carry):
        for par in range(NBUF):
            step(go * NBUF + par, par, fire=True, drain_wb=True)
        return carry

    lax.fori_loop(1, N_CHUNKS // NBUF - 1, outer, 0)

    for par in range(NBUF):
        c = (N_CHUNKS - NBUF) + par
        step(c, par, fire=(par == 0), drain_wb=(par == 0))

    for par in range(NBUF):
        w_copy(N_CHUNKS - NBUF + par, par).wait()


def kernel(x, table):
    tpad = _table_pad_transpose(table.T)
    out5 = _emb_lookup(x.T.astype(jnp.int32), tpad)
    return out5.transpose(2, 4, 0, 1, 3).reshape(B, L, D)


# speedup vs baseline: 2.4648x; 1.0220x over previous
import functools
import math

import jax
import jax.numpy as jnp
from jax import lax
from jax.experimental import pallas as pl
from jax.experimental.pallas import tpu as pltpu
from jax.experimental.pallas import tpu_sc as plsc

VOCAB = 1000000
D = 64
B = 4096
L = 200
N = B * L
DPAD = 128

NC = 2
NS = 16
NW = NC * NS
CHUNK = 128
N_CHUNKS = L
NBUF = 4
TPITCH = 133

SCALE = math.sqrt(D)


TBLK = 24576


def _tpose_body(t_ref, o_ref):
    t = t_ref[...].T
    o_ref[...] = jnp.concatenate([t, t], axis=1)


def _table_pad_transpose(table_t):
    return pl.pallas_call(
        _tpose_body,
        grid=((VOCAB + TBLK - 1) // TBLK,),
        in_specs=[pl.BlockSpec((D, TBLK), lambda i: (0, i))],
        out_specs=pl.BlockSpec((TBLK, DPAD), lambda i: (i, 0)),
        out_shape=jax.ShapeDtypeStruct((VOCAB, DPAD), jnp.float32),
    )(table_t)


_mesh = plsc.VectorSubcoreMesh(core_axis_name="c", subcore_axis_name="s")


@functools.partial(
    pl.kernel,
    mesh=_mesh,
    out_type=jax.ShapeDtypeStruct((L, D // 8, B // CHUNK, 8, CHUNK), jnp.float32),
    scratch_types=[
        pltpu.VMEM((L, CHUNK), jnp.int32),
        pltpu.VMEM((NBUF, CHUNK, DPAD), jnp.float32),
        pltpu.VMEM((NBUF, D // 8, 8, TPITCH), jnp.float32),
    ]
    + [pltpu.SemaphoreType.DMA] * (2 * NBUF),
    compiler_params=pltpu.CompilerParams(
        use_tc_tiling_on_sc=False, needs_layout_passes=False
    ),
)
def _emb_lookup(xt_hbm, table_hbm, out_hbm, idx_v, rows_v, tpose_v, *sems):
    gsems = sems[:NBUF]
    osems = sems[NBUF:]
    wid = lax.axis_index("s") * NC + lax.axis_index("c")
    b0 = wid * CHUNK

    pltpu.sync_copy(xt_hbm.at[:, pl.ds(b0, CHUNK)], idx_v)

    def g_copy(c, b):
        return pltpu.make_async_copy(
            table_hbm.at[idx_v.at[c]],
            rows_v.at[b],
            gsems[b],
        )

    def w_copy(c, b):
        return pltpu.make_async_copy(
            tpose_v.at[b, :, :, pl.ds(0, CHUNK)],
            out_hbm.at[c, :, wid],
            osems[b],
        )

    iota = lax.broadcasted_iota(jnp.int32, (16,), 0)
    dts = [dg * 2 + iota // 8 for dg in range(4)]
    dsv = iota % 8

    def transpose_scale(b):
        @plsc.parallel_loop(0, CHUNK, 1, unroll=8)
        def _t(r):
            rvec = jnp.full((16,), r, jnp.int32)
            for dg in range(4):
                vals = rows_v[b, r, pl.ds(dg * 16, 16)] * jnp.float32(SCALE)
                plsc.store_scatter(tpose_v.at[b], [dts[dg], dsv, rvec], vals)

    def step(c, par, fire, drain_wb):
        b = par
        nb = (par + NBUF - 1) % NBUF
        g_copy(c, b).wait()
        transpose_scale(b)
        w_copy(c, b).start()
        if fire:
            if drain_wb:
                w_copy(c - 1, nb).wait()
            g_copy(c + NBUF - 1, nb).start()

    for c0 in range(NBUF - 1):
        g_copy(c0, c0).start()

    step(0, 0, fire=True, drain_wb=False)
    for par in range(1, NBUF):
        step(par, par, fire=True, drain_wb=True)

    def outer(go, carry):
        for par in range(NBUF):
            step(go * NBUF + par, par, fire=True, drain_wb=True)
        return carry

    lax.fori_loop(1, N_CHUNKS // NBUF - 1, outer, 0)

    for par in range(NBUF):
        c = (N_CHUNKS - NBUF) + par
        step(c, par, fire=(par == 0), drain_wb=(par == 0))

    for par in range(NBUF):
        w_copy(N_CHUNKS - NBUF + par, par).wait()


def kernel(x, table):
    tpad = _table_pad_transpose(table.T)
    out5 = _emb_lookup(x.T.astype(jnp.int32), tpad)
    return out5.transpose(2, 4, 0, 1, 3).reshape(B, L, D)
